# baseline (speedup 1.0000x reference)
import jax
import jax.numpy as jnp
import numpy as np
from jax.experimental import pallas as pl
from jax.experimental.pallas import tpu as pltpu

_N = 256
_R = 3
_P = 2 * _R + 1
_C = 21
_NP = _N + 2 * _R
_S = _N // 8


def _upsample_matrix_padded(src: int, dst: int) -> np.ndarray:
    u = np.zeros((dst + 2 * _R, src), np.float32)
    pos = np.arange(dst, dtype=np.float64) * ((src - 1) / (dst - 1))
    lo = np.floor(pos).astype(np.int64)
    hi = np.minimum(lo + 1, src - 1)
    w = (pos - lo).astype(np.float32)
    u[_R + np.arange(dst), lo] += 1.0 - w
    u[_R + np.arange(dst), hi] += w
    return u


def _pad_matrix(n: int) -> np.ndarray:
    p = np.zeros((n + 2 * _R, n), np.float32)
    p[_R + np.arange(n), np.arange(n)] = 1.0
    return p


def _rwn_kernel(x_ref, x2_ref, c1_ref, c2_ref, w3_ref, w1_ref, w2_ref,
                u32_ref, u32t_ref, u64_ref, u64t_ref, pp_ref, ppt_ref,
                out_ref, a_s, vp_s, den_s):
    f32 = jnp.float32
    f0 = jnp.sum(x_ref[0] * w3_ref[...], axis=0)
    g = (jnp.sum(c1_ref[0] * w1_ref[...], axis=0)
         + jnp.sum(c2_ref[0] * w2_ref[...], axis=0))
    fp = jnp.dot(jnp.dot(pp_ref[...], f0, preferred_element_type=f32),
                 ppt_ref[...], preferred_element_type=f32)
    fp = fp + jnp.dot(jnp.dot(u64_ref[...], g, preferred_element_type=f32),
                      u64t_ref[...], preferred_element_type=f32)
    f = fp[_R:_R + _N, _R:_R + _N]

    t1 = jnp.dot(u32_ref[...], x2_ref[0], preferred_element_type=f32)
    for c in range(_C):
        vp_s[c] = jnp.dot(t1[:, c * _S:(c + 1) * _S], u32t_ref[...],
                          preferred_element_type=f32)

    yi = jax.lax.broadcasted_iota(jnp.int32, (_N, _N), 0)
    xi = jax.lax.broadcasted_iota(jnp.int32, (_N, _N), 1)

    denom = jnp.zeros((_N, _N), f32)
    for dy in range(_P):
        ygood = (yi >= _R - dy) & (yi <= _NP - _R - 1 - dy)
        for dx in range(_P):
            nb = fp[dy:dy + _N, dx:dx + _N]
            good = ygood & (xi >= _R - dx) & (xi <= _NP - _R - 1 - dx)
            a = jnp.where(good, jnp.exp(jnp.abs(f - nb)), 0.0)
            a_s[_P * dy + dx] = a
            denom = denom + a
    den_s[...] = 1.0 / denom

    def body_c(j, _):
        for s in range(4):
            r0 = s * 64
            den = den_s[r0:r0 + 64, :]
            accs = [jnp.zeros((64, _N), f32) for _ in range(3)]
            for dy in range(_P):
                rows = [vp_s[3 * j + i][r0 + dy:r0 + dy + 64, :]
                        for i in range(3)]
                for dx in range(_P):
                    av = a_s[_P * dy + dx, r0:r0 + 64, :]
                    for i in range(3):
                        accs[i] = accs[i] + av * rows[i][:, dx:dx + _N]
            for i in range(3):
                out_ref[0, 3 * j + i, r0:r0 + 64, :] = accs[i] * den
        return 0

    jax.lax.fori_loop(0, _C // 3, body_c, 0)


@jax.jit
def kernel(x, x2feat, conv1, conv2, W_conv):
    B = x.shape[0]
    f32 = jnp.float32
    w = W_conv[0, :, 0, 0]
    w3 = w[:3].reshape(3, 1, 1)
    w1 = w[3:67].reshape(64, 1, 1)
    w2 = w[67:131].reshape(64, 1, 1)
    u32 = jnp.asarray(_upsample_matrix_padded(_S, _N))
    u64 = jnp.asarray(_upsample_matrix_padded(_N // 4, _N))
    pp = jnp.asarray(_pad_matrix(_N))
    x2r = x2feat.transpose(0, 2, 1, 3).reshape(B, _S, _C * _S)

    bspec = lambda shp: pl.BlockSpec(shp, lambda b: (b,) + (0,) * (len(shp) - 1))
    cspec = lambda shp: pl.BlockSpec(shp, lambda b: (0,) * len(shp))

    out = pl.pallas_call(
        _rwn_kernel,
        grid=(B,),
        in_specs=[
            bspec((1, 3, _N, _N)),
            bspec((1, _S, _C * _S)),
            bspec((1, 64, _N // 4, _N // 4)),
            bspec((1, 64, _N // 4, _N // 4)),
            cspec((3, 1, 1)),
            cspec((64, 1, 1)),
            cspec((64, 1, 1)),
            cspec((_NP, _S)),
            cspec((_S, _NP)),
            cspec((_NP, _N // 4)),
            cspec((_N // 4, _NP)),
            cspec((_NP, _N)),
            cspec((_N, _NP)),
        ],
        out_specs=bspec((1, _C, _N, _N)),
        out_shape=jax.ShapeDtypeStruct((B, _C, _N, _N), f32),
        scratch_shapes=[
            pltpu.VMEM((_P * _P, _N, _N), f32),
            pltpu.VMEM((_C, _NP, _NP), f32),
            pltpu.VMEM((_N, _N), f32),
        ],
        compiler_params=pltpu.CompilerParams(
            dimension_semantics=("arbitrary",),
            vmem_limit_bytes=100 * 1024 * 1024,
        ),
    )(x, x2r, conv1, conv2, w3, w1, w2,
      u32, u32.T, u64, u64.T, pp, pp.T)

    return out.reshape(B, _C, _N * _N).transpose(0, 2, 1)


# speedup vs baseline: 1.0333x; 1.0333x over previous
import jax
import jax.numpy as jnp
import numpy as np
from jax.experimental import pallas as pl
from jax.experimental.pallas import tpu as pltpu

_N = 256
_R = 3
_P = 2 * _R + 1
_C = 21
_NP = _N + 2 * _R
_S = _N // 8


def _upsample_matrix_padded(src: int, dst: int) -> np.ndarray:
    u = np.zeros((dst + 2 * _R, src), np.float32)
    pos = np.arange(dst, dtype=np.float64) * ((src - 1) / (dst - 1))
    lo = np.floor(pos).astype(np.int64)
    hi = np.minimum(lo + 1, src - 1)
    w = (pos - lo).astype(np.float32)
    u[_R + np.arange(dst), lo] += 1.0 - w
    u[_R + np.arange(dst), hi] += w
    return u


def _pad_matrix(n: int) -> np.ndarray:
    p = np.zeros((n + 2 * _R, n), np.float32)
    p[_R + np.arange(n), np.arange(n)] = 1.0
    return p


def _rwn_kernel(x_ref, x2_ref, c1_ref, c2_ref, w3_ref, w1_ref, w2_ref,
                u32_ref, u32t_ref, u64_ref, u64t_ref, pp_ref, ppt_ref,
                out_ref, a_s, vp_s, den_s):
    f32 = jnp.float32
    f0 = jnp.sum(x_ref[0] * w3_ref[...], axis=0)
    g = (jnp.sum(c1_ref[0] * w1_ref[...], axis=0)
         + jnp.sum(c2_ref[0] * w2_ref[...], axis=0))
    fp = jnp.dot(jnp.dot(pp_ref[...], f0, preferred_element_type=f32),
                 ppt_ref[...], preferred_element_type=f32)
    fp = fp + jnp.dot(jnp.dot(u64_ref[...], g, preferred_element_type=f32),
                      u64t_ref[...], preferred_element_type=f32)
    f = fp[_R:_R + _N, _R:_R + _N]

    t1 = jnp.dot(u32_ref[...], x2_ref[0], preferred_element_type=f32)
    for c in range(_C):
        vp_s[c] = jnp.dot(t1[:, c * _S:(c + 1) * _S], u32t_ref[...],
                          preferred_element_type=f32)

    yi = jax.lax.broadcasted_iota(jnp.int32, (_N, _N), 0)
    xi = jax.lax.broadcasted_iota(jnp.int32, (_N, _N), 1)

    denom = jnp.zeros((_N, _N), f32)
    for dy in range(_P):
        ygood = (yi >= _R - dy) & (yi <= _NP - _R - 1 - dy)
        for dx in range(_P):
            nb = fp[dy:dy + _N, dx:dx + _N]
            good = ygood & (xi >= _R - dx) & (xi <= _NP - _R - 1 - dx)
            a = jnp.where(good, jnp.exp(jnp.abs(f - nb)), 0.0)
            a_s[_P * dy + dx] = a
            denom = denom + a
    den_s[...] = 1.0 / denom

    def body_c(c, _):
        vp = vp_s[c]
        for s in range(8):
            r0 = s * 32
            acc = jnp.zeros((32, _N), f32)
            for dy in range(_P):
                row = vp[r0 + dy:r0 + dy + 32, :]
                for dx in range(_P):
                    acc = acc + (a_s[_P * dy + dx, r0:r0 + 32, :]
                                 * row[:, dx:dx + _N])
            out_ref[0, c, r0:r0 + 32, :] = acc * den_s[r0:r0 + 32, :]
        return 0

    jax.lax.fori_loop(0, _C, body_c, 0)


@jax.jit
def kernel(x, x2feat, conv1, conv2, W_conv):
    B = x.shape[0]
    f32 = jnp.float32
    w = W_conv[0, :, 0, 0]
    w3 = w[:3].reshape(3, 1, 1)
    w1 = w[3:67].reshape(64, 1, 1)
    w2 = w[67:131].reshape(64, 1, 1)
    u32 = jnp.asarray(_upsample_matrix_padded(_S, _N))
    u64 = jnp.asarray(_upsample_matrix_padded(_N // 4, _N))
    pp = jnp.asarray(_pad_matrix(_N))
    x2r = x2feat.transpose(0, 2, 1, 3).reshape(B, _S, _C * _S)

    bspec = lambda shp: pl.BlockSpec(shp, lambda b: (b,) + (0,) * (len(shp) - 1))
    cspec = lambda shp: pl.BlockSpec(shp, lambda b: (0,) * len(shp))

    out = pl.pallas_call(
        _rwn_kernel,
        grid=(B,),
        in_specs=[
            bspec((1, 3, _N, _N)),
            bspec((1, _S, _C * _S)),
            bspec((1, 64, _N // 4, _N // 4)),
            bspec((1, 64, _N // 4, _N // 4)),
            cspec((3, 1, 1)),
            cspec((64, 1, 1)),
            cspec((64, 1, 1)),
            cspec((_NP, _S)),
            cspec((_S, _NP)),
            cspec((_NP, _N // 4)),
            cspec((_N // 4, _NP)),
            cspec((_NP, _N)),
            cspec((_N, _NP)),
        ],
        out_specs=bspec((1, _C, _N, _N)),
        out_shape=jax.ShapeDtypeStruct((B, _C, _N, _N), f32),
        scratch_shapes=[
            pltpu.VMEM((_P * _P, _N, _N), f32),
            pltpu.VMEM((_C, _NP, _NP), f32),
            pltpu.VMEM((_N, _N), f32),
        ],
        compiler_params=pltpu.CompilerParams(
            dimension_semantics=("arbitrary",),
            vmem_limit_bytes=100 * 1024 * 1024,
        ),
    )(x, x2r, conv1, conv2, w3, w1, w2,
      u32, u32.T, u64, u64.T, pp, pp.T)

    return out.reshape(B, _C, _N * _N).transpose(0, 2, 1)


# speedup vs baseline: 1.0619x; 1.0277x over previous
import jax
import jax.numpy as jnp
import numpy as np
from jax.experimental import pallas as pl
from jax.experimental.pallas import tpu as pltpu

_N = 256
_R = 3
_P = 2 * _R + 1
_C = 21
_NP = _N + 2 * _R
_S = _N // 8


def _upsample_matrix_padded(src: int, dst: int) -> np.ndarray:
    u = np.zeros((dst + 2 * _R, src), np.float32)
    pos = np.arange(dst, dtype=np.float64) * ((src - 1) / (dst - 1))
    lo = np.floor(pos).astype(np.int64)
    hi = np.minimum(lo + 1, src - 1)
    w = (pos - lo).astype(np.float32)
    u[_R + np.arange(dst), lo] += 1.0 - w
    u[_R + np.arange(dst), hi] += w
    return u


def _pad_matrix(n: int) -> np.ndarray:
    p = np.zeros((n + 2 * _R, n), np.float32)
    p[_R + np.arange(n), np.arange(n)] = 1.0
    return p


def _rwn_kernel(x_ref, x2_ref, c1_ref, c2_ref, w3_ref, w1_ref, w2_ref,
                u32_ref, u32t_ref, u64_ref, u64t_ref, pp_ref, ppt_ref,
                out_ref, a_s, vp_s, den_s):
    f32 = jnp.float32
    f0 = jnp.sum(x_ref[0] * w3_ref[...], axis=0)
    g = (jnp.sum(c1_ref[0] * w1_ref[...], axis=0)
         + jnp.sum(c2_ref[0] * w2_ref[...], axis=0))
    fp = jnp.dot(jnp.dot(pp_ref[...], f0, preferred_element_type=f32),
                 ppt_ref[...], preferred_element_type=f32)
    fp = fp + jnp.dot(jnp.dot(u64_ref[...], g, preferred_element_type=f32),
                      u64t_ref[...], preferred_element_type=f32)
    f = fp[_R:_R + _N, _R:_R + _N]

    t1 = jnp.dot(u32_ref[...], x2_ref[0], preferred_element_type=f32)
    for c in range(_C):
        vp_s[c] = jnp.dot(t1[:, c * _S:(c + 1) * _S], u32t_ref[...],
                          preferred_element_type=f32)

    yi = jax.lax.broadcasted_iota(jnp.int32, (_N, _N), 0)
    xi = jax.lax.broadcasted_iota(jnp.int32, (_N, _N), 1)

    denom = jnp.zeros((_N, _N), f32)
    for dy in range(_P):
        ygood = (yi >= _R - dy) & (yi <= _NP - _R - 1 - dy)
        for dx in range(_P):
            nb = fp[dy:dy + _N, dx:dx + _N]
            good = ygood & (xi >= _R - dx) & (xi <= _NP - _R - 1 - dx)
            a = jnp.where(good, jnp.exp(jnp.abs(f - nb)), 0.0)
            a_s[_P * dy + dx] = a
            denom = denom + a
    den_s[...] = 1.0 / denom

    def body_c(c, _):
        vp = vp_s[c]
        for s in range(16):
            r0 = s * 16
            acc = jnp.zeros((16, _N), f32)
            for dy in range(_P):
                row = vp[r0 + dy:r0 + dy + 16, :]
                for dx in range(_P):
                    acc = acc + (a_s[_P * dy + dx, r0:r0 + 16, :]
                                 * row[:, dx:dx + _N])
            out_ref[0, c, r0:r0 + 16, :] = acc * den_s[r0:r0 + 16, :]
        return 0

    jax.lax.fori_loop(0, _C, body_c, 0)


@jax.jit
def kernel(x, x2feat, conv1, conv2, W_conv):
    B = x.shape[0]
    f32 = jnp.float32
    w = W_conv[0, :, 0, 0]
    w3 = w[:3].reshape(3, 1, 1)
    w1 = w[3:67].reshape(64, 1, 1)
    w2 = w[67:131].reshape(64, 1, 1)
    u32 = jnp.asarray(_upsample_matrix_padded(_S, _N))
    u64 = jnp.asarray(_upsample_matrix_padded(_N // 4, _N))
    pp = jnp.asarray(_pad_matrix(_N))
    x2r = x2feat.transpose(0, 2, 1, 3).reshape(B, _S, _C * _S)

    bspec = lambda shp: pl.BlockSpec(shp, lambda b: (b,) + (0,) * (len(shp) - 1))
    cspec = lambda shp: pl.BlockSpec(shp, lambda b: (0,) * len(shp))

    out = pl.pallas_call(
        _rwn_kernel,
        grid=(B,),
        in_specs=[
            bspec((1, 3, _N, _N)),
            bspec((1, _S, _C * _S)),
            bspec((1, 64, _N // 4, _N // 4)),
            bspec((1, 64, _N // 4, _N // 4)),
            cspec((3, 1, 1)),
            cspec((64, 1, 1)),
            cspec((64, 1, 1)),
            cspec((_NP, _S)),
            cspec((_S, _NP)),
            cspec((_NP, _N // 4)),
            cspec((_N // 4, _NP)),
            cspec((_NP, _N)),
            cspec((_N, _NP)),
        ],
        out_specs=bspec((1, _C, _N, _N)),
        out_shape=jax.ShapeDtypeStruct((B, _C, _N, _N), f32),
        scratch_shapes=[
            pltpu.VMEM((_P * _P, _N, _N), f32),
            pltpu.VMEM((_C, _NP, _NP), f32),
            pltpu.VMEM((_N, _N), f32),
        ],
        compiler_params=pltpu.CompilerParams(
            dimension_semantics=("arbitrary",),
            vmem_limit_bytes=100 * 1024 * 1024,
        ),
    )(x, x2r, conv1, conv2, w3, w1, w2,
      u32, u32.T, u64, u64.T, pp, pp.T)

    return out.reshape(B, _C, _N * _N).transpose(0, 2, 1)


# speedup vs baseline: 1.0852x; 1.0219x over previous
import jax
import jax.numpy as jnp
import numpy as np
from jax.experimental import pallas as pl
from jax.experimental.pallas import tpu as pltpu

_N = 256
_R = 3
_P = 2 * _R + 1
_C = 21
_NP = _N + 2 * _R
_S = _N // 8


def _upsample_matrix_padded(src: int, dst: int) -> np.ndarray:
    u = np.zeros((dst + 2 * _R, src), np.float32)
    pos = np.arange(dst, dtype=np.float64) * ((src - 1) / (dst - 1))
    lo = np.floor(pos).astype(np.int64)
    hi = np.minimum(lo + 1, src - 1)
    w = (pos - lo).astype(np.float32)
    u[_R + np.arange(dst), lo] += 1.0 - w
    u[_R + np.arange(dst), hi] += w
    return u


def _pad_matrix(n: int) -> np.ndarray:
    p = np.zeros((n + 2 * _R, n), np.float32)
    p[_R + np.arange(n), np.arange(n)] = 1.0
    return p


def _rwn_kernel(x_ref, x2_ref, c1_ref, c2_ref, w3_ref, w1_ref, w2_ref,
                u32_ref, u32t_ref, u64_ref, u64t_ref, pp_ref, ppt_ref,
                out_ref, a_s, vp_s, den_s):
    f32 = jnp.float32
    f0 = jnp.sum(x_ref[0] * w3_ref[...], axis=0)
    g = (jnp.sum(c1_ref[0] * w1_ref[...], axis=0)
         + jnp.sum(c2_ref[0] * w2_ref[...], axis=0))
    fp = jnp.dot(jnp.dot(pp_ref[...], f0, preferred_element_type=f32),
                 ppt_ref[...], preferred_element_type=f32)
    fp = fp + jnp.dot(jnp.dot(u64_ref[...], g, preferred_element_type=f32),
                      u64t_ref[...], preferred_element_type=f32)
    f = fp[_R:_R + _N, _R:_R + _N]

    t1 = jnp.dot(u32_ref[...], x2_ref[0], preferred_element_type=f32)
    for c in range(_C):
        vp_s[c] = jnp.dot(t1[:, c * _S:(c + 1) * _S], u32t_ref[...],
                          preferred_element_type=f32)

    yi = jax.lax.broadcasted_iota(jnp.int32, (_N, _N), 0)
    xi = jax.lax.broadcasted_iota(jnp.int32, (_N, _N), 1)

    denom = jnp.zeros((_N, _N), f32)
    for dy in range(_P):
        ygood = (yi >= _R - dy) & (yi <= _NP - _R - 1 - dy)
        for dx in range(_P):
            nb = fp[dy:dy + _N, dx:dx + _N]
            good = ygood & (xi >= _R - dx) & (xi <= _NP - _R - 1 - dx)
            a = jnp.where(good, jnp.exp(jnp.abs(f - nb)), 0.0)
            a_s[_P * dy + dx] = a
            denom = denom + a
    den_s[...] = 1.0 / denom

    def body_c(c, _):
        vp = vp_s[c]
        for s in range(32):
            r0 = s * 8
            acc = jnp.zeros((8, _N), f32)
            for dy in range(_P):
                row = vp[r0 + dy:r0 + dy + 8, :]
                for dx in range(_P):
                    acc = acc + (a_s[_P * dy + dx, r0:r0 + 8, :]
                                 * row[:, dx:dx + _N])
            out_ref[0, c, r0:r0 + 8, :] = acc * den_s[r0:r0 + 8, :]
        return 0

    jax.lax.fori_loop(0, _C, body_c, 0)


@jax.jit
def kernel(x, x2feat, conv1, conv2, W_conv):
    B = x.shape[0]
    f32 = jnp.float32
    w = W_conv[0, :, 0, 0]
    w3 = w[:3].reshape(3, 1, 1)
    w1 = w[3:67].reshape(64, 1, 1)
    w2 = w[67:131].reshape(64, 1, 1)
    u32 = jnp.asarray(_upsample_matrix_padded(_S, _N))
    u64 = jnp.asarray(_upsample_matrix_padded(_N // 4, _N))
    pp = jnp.asarray(_pad_matrix(_N))
    x2r = x2feat.transpose(0, 2, 1, 3).reshape(B, _S, _C * _S)

    bspec = lambda shp: pl.BlockSpec(shp, lambda b: (b,) + (0,) * (len(shp) - 1))
    cspec = lambda shp: pl.BlockSpec(shp, lambda b: (0,) * len(shp))

    out = pl.pallas_call(
        _rwn_kernel,
        grid=(B,),
        in_specs=[
            bspec((1, 3, _N, _N)),
            bspec((1, _S, _C * _S)),
            bspec((1, 64, _N // 4, _N // 4)),
            bspec((1, 64, _N // 4, _N // 4)),
            cspec((3, 1, 1)),
            cspec((64, 1, 1)),
            cspec((64, 1, 1)),
            cspec((_NP, _S)),
            cspec((_S, _NP)),
            cspec((_NP, _N // 4)),
            cspec((_N // 4, _NP)),
            cspec((_NP, _N)),
            cspec((_N, _NP)),
        ],
        out_specs=bspec((1, _C, _N, _N)),
        out_shape=jax.ShapeDtypeStruct((B, _C, _N, _N), f32),
        scratch_shapes=[
            pltpu.VMEM((_P * _P, _N, _N), f32),
            pltpu.VMEM((_C, _NP, _NP), f32),
            pltpu.VMEM((_N, _N), f32),
        ],
        compiler_params=pltpu.CompilerParams(
            dimension_semantics=("arbitrary",),
            vmem_limit_bytes=100 * 1024 * 1024,
        ),
    )(x, x2r, conv1, conv2, w3, w1, w2,
      u32, u32.T, u64, u64.T, pp, pp.T)

    return out.reshape(B, _C, _N * _N).transpose(0, 2, 1)
